# Initial kernel scaffold; baseline (speedup 1.0000x reference)
import jax
import jax.numpy as jnp
from jax.experimental import pallas as pl


def kernel(x, emb_weight, lin_weight, lin_bias):
    raise NotImplementedError("write your pallas kernel here")


# speedup vs baseline: 19.6756x; 19.6756x over previous
import functools

import jax
import jax.numpy as jnp
from jax import lax
from jax.experimental import pallas as pl
from jax.experimental.pallas import tpu as pltpu
from jax.experimental.pallas import tpu_sc as plsc

VOCAB = 1000
EMB = 100
B = 16384
L = 200

NC = 2
NS = 16
NW = NC * NS
ROWS_PER_W = B // NW
CHUNK = 16
CHUNKS_PER_W = ROWS_PER_W // CHUNK
XWORDS = CHUNK * L
CWORDS = CHUNK * VOCAB


def _sc_counts_body(xt_hbm, counts_hbm, x_v, cnt_v):
    wid = lax.axis_index("s") * NC + lax.axis_index("c")
    lane = lax.iota(jnp.int32, 16)
    row_off = lane * VOCAB
    ones = jnp.full((16,), 1.0, jnp.float32)
    zeros = jnp.zeros((16,), jnp.float32)

    def chunk_body(k, _):
        c = wid * CHUNKS_PER_W + k
        pltpu.sync_copy(xt_hbm.at[pl.ds(c * XWORDS, XWORDS)], x_v)

        def zbody(i, _):
            cnt_v[pl.ds(i * 16, 16)] = zeros
            return 0

        lax.fori_loop(0, CWORDS // 16, zbody, 0)

        def lbody(l, _):
            vals = x_v[pl.ds(l * 16, 16)]
            plsc.addupdate_scatter(cnt_v, [row_off + vals], ones)
            return 0

        lax.fori_loop(0, L, lbody, 0)
        pltpu.sync_copy(cnt_v, counts_hbm.at[pl.ds(c * CWORDS, CWORDS)])
        return 0

    lax.fori_loop(0, CHUNKS_PER_W, chunk_body, 0)


_sc_counts = pl.kernel(
    _sc_counts_body,
    out_type=jax.ShapeDtypeStruct((B * VOCAB,), jnp.float32),
    mesh=plsc.VectorSubcoreMesh(core_axis_name="c", subcore_axis_name="s"),
    scratch_types=[
        pltpu.VMEM((XWORDS,), jnp.int32),
        pltpu.VMEM((CWORDS,), jnp.float32),
    ],
    compiler_params=pltpu.CompilerParams(needs_layout_passes=False),
)


BLK = 512


def _tc_body(counts_ref, emb_ref, lin_ref, bias_ref, out_ref):
    cnt = counts_ref[...]
    vec = jnp.dot(cnt, emb_ref[...], preferred_element_type=jnp.float32,
                  precision=lax.Precision.HIGHEST)
    vec = vec * jnp.float32(1.0 / L)
    out = lax.dot_general(vec, lin_ref[...], (((1,), (1,)), ((), ())),
                          preferred_element_type=jnp.float32,
                          precision=lax.Precision.HIGHEST)
    out_ref[...] = out + bias_ref[...]


_tc_linear = pl.pallas_call(
    _tc_body,
    grid=(B // BLK,),
    in_specs=[
        pl.BlockSpec((BLK, VOCAB), lambda i: (i, 0)),
        pl.BlockSpec((VOCAB, EMB), lambda i: (0, 0)),
        pl.BlockSpec((VOCAB, EMB), lambda i: (0, 0)),
        pl.BlockSpec((1, VOCAB), lambda i: (0, 0)),
    ],
    out_specs=pl.BlockSpec((BLK, VOCAB), lambda i: (i, 0)),
    out_shape=jax.ShapeDtypeStruct((B, VOCAB), jnp.float32),
)


@jax.jit
def kernel(x, emb_weight, lin_weight, lin_bias):
    xt = x.reshape(B // CHUNK, CHUNK, L).transpose(0, 2, 1).reshape(-1)
    counts = _sc_counts(xt).reshape(B, VOCAB)
    return _tc_linear(counts, emb_weight, lin_weight, lin_bias.reshape(1, VOCAB))


# speedup vs baseline: 32.5723x; 1.6555x over previous
import functools

import jax
import jax.numpy as jnp
from jax import lax
from jax.experimental import pallas as pl
from jax.experimental.pallas import tpu as pltpu
from jax.experimental.pallas import tpu_sc as plsc

VOCAB = 1000
EMB = 100
B = 16384
L = 200

NC = 2
NS = 16
NW = NC * NS
ROWS_PER_W = B // NW
CHUNK = 16
CHUNKS_PER_W = ROWS_PER_W // CHUNK
XWORDS = CHUNK * L
CWORDS = CHUNK * VOCAB


def _sc_counts_body(xt_hbm, counts_hbm, x_v, cnt_v):
    wid = lax.axis_index("s") * NC + lax.axis_index("c")
    lane = lax.iota(jnp.int32, 16)
    row_off = lane * VOCAB
    ones = jnp.full((16,), 1.0, jnp.float32)
    zeros = jnp.zeros((16,), jnp.float32)

    def chunk_body(k, _):
        c = wid * CHUNKS_PER_W + k
        pltpu.sync_copy(xt_hbm.at[pl.ds(c * XWORDS, XWORDS)], x_v)

        def zbody(i, _):
            cnt_v[pl.ds(i * 16, 16)] = zeros
            return 0

        lax.fori_loop(0, CWORDS // 16, zbody, 0, unroll=10)

        def lbody(l, _):
            vals = x_v[pl.ds(l * 16, 16)]
            plsc.addupdate_scatter(cnt_v, [row_off + vals], ones)
            return 0

        lax.fori_loop(0, L, lbody, 0, unroll=10)
        pltpu.sync_copy(cnt_v, counts_hbm.at[pl.ds(c * CWORDS, CWORDS)])
        return 0

    lax.fori_loop(0, CHUNKS_PER_W, chunk_body, 0)


_sc_counts = pl.kernel(
    _sc_counts_body,
    out_type=jax.ShapeDtypeStruct((B * VOCAB,), jnp.float32),
    mesh=plsc.VectorSubcoreMesh(core_axis_name="c", subcore_axis_name="s"),
    scratch_types=[
        pltpu.VMEM((XWORDS,), jnp.int32),
        pltpu.VMEM((CWORDS,), jnp.float32),
    ],
    compiler_params=pltpu.CompilerParams(needs_layout_passes=False),
)


BLK = 512


def _tc_body(counts_ref, emb_ref, lin_ref, bias_ref, out_ref):
    cnt = counts_ref[...]
    vec = jnp.dot(cnt, emb_ref[...], preferred_element_type=jnp.float32)
    vec = vec * jnp.float32(1.0 / L)
    out = lax.dot_general(vec, lin_ref[...], (((1,), (1,)), ((), ())),
                          preferred_element_type=jnp.float32)
    out_ref[...] = out + bias_ref[...]


_tc_linear = pl.pallas_call(
    _tc_body,
    grid=(B // BLK,),
    in_specs=[
        pl.BlockSpec((BLK, VOCAB), lambda i: (i, 0)),
        pl.BlockSpec((VOCAB, EMB), lambda i: (0, 0)),
        pl.BlockSpec((VOCAB, EMB), lambda i: (0, 0)),
        pl.BlockSpec((1, VOCAB), lambda i: (0, 0)),
    ],
    out_specs=pl.BlockSpec((BLK, VOCAB), lambda i: (i, 0)),
    out_shape=jax.ShapeDtypeStruct((B, VOCAB), jnp.float32),
)


@jax.jit
def kernel(x, emb_weight, lin_weight, lin_bias):
    xt = x.reshape(B // CHUNK, CHUNK, L).transpose(0, 2, 1).reshape(-1)
    counts = _sc_counts(xt).reshape(B, VOCAB)
    return _tc_linear(counts, emb_weight, lin_weight, lin_bias.reshape(1, VOCAB))


# speedup vs baseline: 36.7597x; 1.1286x over previous
import functools

import jax
import jax.numpy as jnp
from jax import lax
from jax.experimental import pallas as pl
from jax.experimental.pallas import tpu as pltpu
from jax.experimental.pallas import tpu_sc as plsc

VOCAB = 1000
EMB = 100
B = 16384
L = 200

NC = 2
NS = 16
NW = NC * NS
ROWS_PER_W = B // NW
CHUNK = 16
CHUNKS_PER_W = ROWS_PER_W // CHUNK
XWORDS = CHUNK * L
CWORDS = CHUNK * VOCAB


def _sc_counts_body(xt_hbm, counts_hbm, x0, x1, c0, c1, si0, si1, so0, so1):
    wid = lax.axis_index("s") * NC + lax.axis_index("c")
    lane = lax.iota(jnp.int32, 16)
    row_off = lane * VOCAB
    ones = jnp.full((16,), 1.0, jnp.float32)
    zeros = jnp.zeros((16,), jnp.float32)
    base = wid * CHUNKS_PER_W

    def issue_in(k, xbuf, sem):
        pltpu.async_copy(xt_hbm.at[pl.ds((base + k) * XWORDS, XWORDS)], xbuf, sem)

    def process(k, xbuf, cbuf, in_sem, out_sem, first):
        if not first:
            pltpu.make_async_copy(cnt_dummy_src, cbuf, out_sem).wait()

        def zbody(i, _):
            cbuf[pl.ds(i * 16, 16)] = zeros
            return 0

        lax.fori_loop(0, CWORDS // 16, zbody, 0, unroll=10)
        pltpu.make_async_copy(x_dummy_src, xbuf, in_sem).wait()

        def lbody(l, _):
            vals = xbuf[pl.ds(l * 16, 16)]
            plsc.addupdate_scatter(cbuf, [row_off + vals], ones)
            return 0

        lax.fori_loop(0, L, lbody, 0, unroll=10)
        pltpu.async_copy(cbuf, counts_hbm.at[pl.ds((base + k) * CWORDS, CWORDS)],
                         out_sem)

    x_dummy_src = xt_hbm.at[pl.ds(0, XWORDS)]
    cnt_dummy_src = counts_hbm.at[pl.ds(0, CWORDS)]

    issue_in(0, x0, si0)
    issue_in(1, x1, si1)
    process(0, x0, c0, si0, so0, first=True)
    issue_in(2, x0, si0)
    process(1, x1, c1, si1, so1, first=True)
    issue_in(3, x1, si1)

    def pair_body(i, _):
        k = 2 * i
        process(k, x0, c0, si0, so0, first=False)

        @pl.when(i < CHUNKS_PER_W // 2 - 1)
        def _():
            issue_in(k + 2, x0, si0)

        process(k + 1, x1, c1, si1, so1, first=False)

        @pl.when(i < CHUNKS_PER_W // 2 - 1)
        def _():
            issue_in(k + 3, x1, si1)

        return 0

    lax.fori_loop(1, CHUNKS_PER_W // 2, pair_body, 0)

    pltpu.make_async_copy(cnt_dummy_src, c0, so0).wait()
    pltpu.make_async_copy(cnt_dummy_src, c1, so1).wait()


_sc_counts = pl.kernel(
    _sc_counts_body,
    out_type=jax.ShapeDtypeStruct((B * VOCAB,), jnp.float32),
    mesh=plsc.VectorSubcoreMesh(core_axis_name="c", subcore_axis_name="s"),
    scratch_types=[
        pltpu.VMEM((XWORDS,), jnp.int32),
        pltpu.VMEM((XWORDS,), jnp.int32),
        pltpu.VMEM((CWORDS,), jnp.float32),
        pltpu.VMEM((CWORDS,), jnp.float32),
        pltpu.SemaphoreType.DMA,
        pltpu.SemaphoreType.DMA,
        pltpu.SemaphoreType.DMA,
        pltpu.SemaphoreType.DMA,
    ],
    compiler_params=pltpu.CompilerParams(needs_layout_passes=False),
)


BLK = 512


def _tc_body(counts_ref, emb_ref, lin_ref, bias_ref, out_ref):
    cnt = counts_ref[...]
    vec = jnp.dot(cnt, emb_ref[...], preferred_element_type=jnp.float32)
    vec = vec * jnp.float32(1.0 / L)
    out = lax.dot_general(vec, lin_ref[...], (((1,), (1,)), ((), ())),
                          preferred_element_type=jnp.float32)
    out_ref[...] = out + bias_ref[...]


_tc_linear = pl.pallas_call(
    _tc_body,
    grid=(B // BLK,),
    in_specs=[
        pl.BlockSpec((BLK, VOCAB), lambda i: (i, 0)),
        pl.BlockSpec((VOCAB, EMB), lambda i: (0, 0)),
        pl.BlockSpec((VOCAB, EMB), lambda i: (0, 0)),
        pl.BlockSpec((1, VOCAB), lambda i: (0, 0)),
    ],
    out_specs=pl.BlockSpec((BLK, VOCAB), lambda i: (i, 0)),
    out_shape=jax.ShapeDtypeStruct((B, VOCAB), jnp.float32),
)


@jax.jit
def kernel(x, emb_weight, lin_weight, lin_bias):
    xt = x.reshape(B // CHUNK, CHUNK, L).transpose(0, 2, 1).reshape(-1)
    counts = _sc_counts(xt).reshape(B, VOCAB)
    return _tc_linear(counts, emb_weight, lin_weight, lin_bias.reshape(1, VOCAB))


# speedup vs baseline: 59.1265x; 1.6085x over previous
import functools

import jax
import jax.numpy as jnp
from jax import lax
from jax.experimental import pallas as pl
from jax.experimental.pallas import tpu as pltpu
from jax.experimental.pallas import tpu_sc as plsc

VOCAB = 1000
EMB = 100
B = 16384
L = 200

NC = 2
NS = 16
NW = NC * NS
ROWS_PER_W = B // NW
CHUNK = 16
CHUNKS_PER_W = ROWS_PER_W // CHUNK
NJ = L // 16
LREM = L - NJ * 16


def _sc_counts_body(x_hbm, counts_hbm, x0, x1, c0, c1, si0, si1, so0, so1):
    wid = lax.axis_index("s") * NC + lax.axis_index("c")
    lane = lax.iota(jnp.int32, 16)
    tail_mask = lane >= (16 - LREM)
    ones = jnp.full((16,), 1.0, jnp.float32)
    zeros = jnp.zeros((16,), jnp.float32)
    base = wid * CHUNKS_PER_W

    def issue_in(k, xbuf, sem):
        pltpu.async_copy(x_hbm.at[pl.ds((base + k) * CHUNK, CHUNK), :], xbuf, sem)

    def process(k, xbuf, cbuf, in_sem, out_sem, first):
        if not first:
            pltpu.make_async_copy(cnt_dummy_src, cbuf, out_sem).wait()

        def zrow(r, _):
            def zbody(i, _):
                cbuf[r, pl.ds(i * 16, 16)] = zeros
                return 0

            lax.fori_loop(0, VOCAB // 16, zbody, 0, unroll=10)
            cbuf[r, pl.ds(VOCAB - 16, 16)] = zeros
            return 0

        lax.fori_loop(0, CHUNK, zrow, 0)

        pltpu.make_async_copy(x_dummy_src, xbuf, in_sem).wait()

        def srow(r, _):
            rsplat = jnp.full((16,), r, jnp.int32)

            def jbody(j, _):
                vals = xbuf[r, pl.ds(j * 16, 16)]
                plsc.addupdate_scatter(cbuf, [rsplat, vals], ones)
                return 0

            lax.fori_loop(0, NJ, jbody, 0, unroll=NJ)
            vals = xbuf[r, pl.ds(L - 16, 16)]
            plsc.addupdate_scatter(cbuf, [rsplat, vals], ones, mask=tail_mask)
            return 0

        lax.fori_loop(0, CHUNK, srow, 0)

        pltpu.async_copy(cbuf, counts_hbm.at[pl.ds((base + k) * CHUNK, CHUNK), :],
                         out_sem)

    x_dummy_src = x_hbm.at[pl.ds(0, CHUNK), :]
    cnt_dummy_src = counts_hbm.at[pl.ds(0, CHUNK), :]

    issue_in(0, x0, si0)
    issue_in(1, x1, si1)
    process(0, x0, c0, si0, so0, first=True)
    issue_in(2, x0, si0)
    process(1, x1, c1, si1, so1, first=True)
    issue_in(3, x1, si1)

    def pair_body(i, _):
        k = 2 * i
        process(k, x0, c0, si0, so0, first=False)

        @pl.when(i < CHUNKS_PER_W // 2 - 1)
        def _():
            issue_in(k + 2, x0, si0)

        process(k + 1, x1, c1, si1, so1, first=False)

        @pl.when(i < CHUNKS_PER_W // 2 - 1)
        def _():
            issue_in(k + 3, x1, si1)

        return 0

    lax.fori_loop(1, CHUNKS_PER_W // 2, pair_body, 0)

    pltpu.make_async_copy(cnt_dummy_src, c0, so0).wait()
    pltpu.make_async_copy(cnt_dummy_src, c1, so1).wait()


_sc_counts = pl.kernel(
    _sc_counts_body,
    out_type=jax.ShapeDtypeStruct((B, VOCAB), jnp.float32),
    mesh=plsc.VectorSubcoreMesh(core_axis_name="c", subcore_axis_name="s"),
    scratch_types=[
        pltpu.VMEM((CHUNK, L), jnp.int32),
        pltpu.VMEM((CHUNK, L), jnp.int32),
        pltpu.VMEM((CHUNK, VOCAB), jnp.float32),
        pltpu.VMEM((CHUNK, VOCAB), jnp.float32),
        pltpu.SemaphoreType.DMA,
        pltpu.SemaphoreType.DMA,
        pltpu.SemaphoreType.DMA,
        pltpu.SemaphoreType.DMA,
    ],
    compiler_params=pltpu.CompilerParams(needs_layout_passes=False),
)


BLK = 512


def _tc_body(counts_ref, emb_ref, lin_ref, bias_ref, out_ref):
    cnt = counts_ref[...]
    vec = jnp.dot(cnt, emb_ref[...], preferred_element_type=jnp.float32)
    vec = vec * jnp.float32(1.0 / L)
    out = lax.dot_general(vec, lin_ref[...], (((1,), (1,)), ((), ())),
                          preferred_element_type=jnp.float32)
    out_ref[...] = out + bias_ref[...]


_tc_linear = pl.pallas_call(
    _tc_body,
    grid=(B // BLK,),
    in_specs=[
        pl.BlockSpec((BLK, VOCAB), lambda i: (i, 0)),
        pl.BlockSpec((VOCAB, EMB), lambda i: (0, 0)),
        pl.BlockSpec((VOCAB, EMB), lambda i: (0, 0)),
        pl.BlockSpec((1, VOCAB), lambda i: (0, 0)),
    ],
    out_specs=pl.BlockSpec((BLK, VOCAB), lambda i: (i, 0)),
    out_shape=jax.ShapeDtypeStruct((B, VOCAB), jnp.float32),
)


@jax.jit
def kernel(x, emb_weight, lin_weight, lin_bias):
    counts = _sc_counts(x)
    return _tc_linear(counts, emb_weight, lin_weight, lin_bias.reshape(1, VOCAB))


# speedup vs baseline: 81.1036x; 1.3717x over previous
import functools

import jax
import jax.numpy as jnp
from jax import lax
from jax.experimental import pallas as pl
from jax.experimental.pallas import tpu as pltpu
from jax.experimental.pallas import tpu_sc as plsc

VOCAB = 1000
EMB = 100
B = 16384
L = 200

NC = 2
NS = 16
NW = NC * NS
ROWS_PER_W = B // NW
CHUNK = 16
CHUNKS_PER_W = ROWS_PER_W // CHUNK


XBLK = 128
NXB = ROWS_PER_W // XBLK
GRP = XBLK // CHUNK


def _sc_counts_body(xt_hbm, counts_hbm, xa, xb, c0, c1, sxa, sxb, so0, so1):
    wid = lax.axis_index("s") * NC + lax.axis_index("c")
    lane = lax.iota(jnp.int32, 16)
    ones = jnp.full((16,), 1.0, jnp.float32)
    zeros = jnp.zeros((16,), jnp.float32)
    rbase = wid * ROWS_PER_W

    xbufs = [(xa, sxa), (xb, sxb)]
    cbufs = [(c0, so0), (c1, so1)]
    x_dummy = xt_hbm.at[:, pl.ds(0, XBLK)]
    cnt_dummy = counts_hbm.at[pl.ds(0, CHUNK), :]

    def issue_x(t, buf, sem):
        pltpu.async_copy(xt_hbm.at[:, pl.ds(rbase + t * XBLK, XBLK)], buf, sem)

    issue_x(0, xa, sxa)
    issue_x(1, xb, sxb)

    for t in range(NXB):
        xbuf, xsem = xbufs[t % 2]
        pltpu.make_async_copy(x_dummy, xbuf, xsem).wait()

        for g in range(GRP):
            kk = t * GRP + g
            cbuf, osem = cbufs[kk % 2]
            if kk >= 2:
                pltpu.make_async_copy(cnt_dummy, cbuf, osem).wait()

            def zrow(r, _, cbuf=cbuf):
                def zbody(i, _):
                    cbuf[r, pl.ds(i * 16, 16)] = zeros
                    return 0

                lax.fori_loop(0, VOCAB // 16, zbody, 0, unroll=10)
                cbuf[r, pl.ds(VOCAB - 16, 16)] = zeros
                return 0

            lax.fori_loop(0, CHUNK, zrow, 0)

            def lbody(l, _, cbuf=cbuf, xbuf=xbuf, g=g):
                vals = xbuf[l, pl.ds(g * CHUNK, CHUNK)]
                plsc.addupdate_scatter(cbuf, [lane, vals], ones)
                return 0

            lax.fori_loop(0, L, lbody, 0, unroll=10)

            row0 = rbase + t * XBLK + g * CHUNK
            pltpu.async_copy(cbuf, counts_hbm.at[pl.ds(row0, CHUNK), :], osem)

        if t + 2 < NXB:
            issue_x(t + 2, xbuf, xsem)

    pltpu.make_async_copy(cnt_dummy, c0, so0).wait()
    pltpu.make_async_copy(cnt_dummy, c1, so1).wait()


_sc_counts = pl.kernel(
    _sc_counts_body,
    out_type=jax.ShapeDtypeStruct((B, VOCAB), jnp.float32),
    mesh=plsc.VectorSubcoreMesh(core_axis_name="c", subcore_axis_name="s"),
    scratch_types=[
        pltpu.VMEM((L, XBLK), jnp.int32),
        pltpu.VMEM((L, XBLK), jnp.int32),
        pltpu.VMEM((CHUNK, VOCAB), jnp.float32),
        pltpu.VMEM((CHUNK, VOCAB), jnp.float32),
        pltpu.SemaphoreType.DMA,
        pltpu.SemaphoreType.DMA,
        pltpu.SemaphoreType.DMA,
        pltpu.SemaphoreType.DMA,
    ],
    compiler_params=pltpu.CompilerParams(needs_layout_passes=False),
)


BLK = 512


def _tc_body(counts_ref, embt_ref, lint_ref, bias_ref, outt_ref):
    cnt = counts_ref[...]
    vec = lax.dot_general(cnt, embt_ref[...], (((1,), (1,)), ((), ())),
                          preferred_element_type=jnp.float32)
    vec = vec * jnp.float32(1.0 / L)
    outt = lax.dot_general(lint_ref[...], vec, (((0,), (1,)), ((), ())),
                           preferred_element_type=jnp.float32)
    outt_ref[...] = outt + bias_ref[...]


_tc_linear = pl.pallas_call(
    _tc_body,
    grid=(B // BLK,),
    in_specs=[
        pl.BlockSpec((BLK, VOCAB), lambda i: (i, 0)),
        pl.BlockSpec((EMB, VOCAB), lambda i: (0, 0)),
        pl.BlockSpec((EMB, VOCAB), lambda i: (0, 0)),
        pl.BlockSpec((VOCAB, 1), lambda i: (0, 0)),
    ],
    out_specs=pl.BlockSpec((VOCAB, BLK), lambda i: (0, i)),
    out_shape=jax.ShapeDtypeStruct((VOCAB, B), jnp.float32),
)


@jax.jit
def kernel(x, emb_weight, lin_weight, lin_bias):
    counts = _sc_counts(x.T)
    predt = _tc_linear(counts, emb_weight.T, lin_weight.T,
                       lin_bias.reshape(VOCAB, 1))
    return predt.T
